# Initial kernel scaffold; baseline (speedup 1.0000x reference)
import jax
import jax.numpy as jnp
from jax.experimental import pallas as pl


def kernel(inputs, Wg, W1, b1, W2, b2):
    raise NotImplementedError("write your pallas kernel here")


# speedup vs baseline: 1.0531x; 1.0531x over previous
import functools
import jax
import jax.numpy as jnp
from jax.experimental import pallas as pl
from jax.experimental.pallas import tpu as pltpu

_E = 16
_D_MODEL = 768
_D_FF = 2048
_T = 128
_F = 512
_NF = _D_FF // _F


def _gate_body(x_ref, wg_ref, eid_ref):
    logits = jnp.dot(x_ref[...], wg_ref[...], preferred_element_type=jnp.float32)
    m = jnp.max(logits, axis=-1, keepdims=True)
    s = jnp.exp(logits - m)
    p = s / jnp.sum(s, axis=-1, keepdims=True)
    eid_ref[0, 0, :] = jnp.argmax(p, axis=-1).astype(jnp.int32)


def _gate(flat, Wg):
    n = flat.shape[0]
    nblk = n // _T
    eid2d = pl.pallas_call(
        _gate_body,
        grid=(nblk,),
        in_specs=[
            pl.BlockSpec((_T, _D_MODEL), lambda i: (i, 0)),
            pl.BlockSpec((_D_MODEL, _E), lambda i: (0, 0)),
        ],
        out_specs=pl.BlockSpec((1, 1, _T), lambda i: (i, 0, 0)),
        out_shape=jax.ShapeDtypeStruct((nblk, 1, _T), jnp.int32),
    )(flat, Wg)
    return eid2d.reshape(-1)


def _moe_body(eids_ref, meta_ref, x_ref, w1_ref, b1_ref, w2_ref, b2_ref, o_ref):
    g = pl.program_id(0)
    f = pl.program_id(1)

    @pl.when(f == 0)
    def _():
        o_ref[...] = jnp.zeros_like(o_ref)

    @pl.when(g < meta_ref[0])
    def _():
        h = jnp.dot(x_ref[...], w1_ref[0], preferred_element_type=jnp.float32)
        h = jnp.maximum(h + b1_ref[0], 0.0)
        o_ref[...] += jnp.dot(h, w2_ref[0], preferred_element_type=jnp.float32)

    @pl.when(jnp.logical_and(f == _NF - 1, g < meta_ref[0]))
    def _():
        o_ref[...] += b2_ref[0]


def _grouped_mlp(x_pad, tile_eid, ntiles, W1, b1, W2, b2, g_max):
    grid_spec = pltpu.PrefetchScalarGridSpec(
        num_scalar_prefetch=2,
        grid=(g_max, _NF),
        in_specs=[
            pl.BlockSpec((_T, _D_MODEL), lambda g, f, e, m: (g, 0)),
            pl.BlockSpec((1, _D_MODEL, _F), lambda g, f, e, m: (e[g], 0, f)),
            pl.BlockSpec((1, 1, _F), lambda g, f, e, m: (e[g], 0, f)),
            pl.BlockSpec((1, _F, _D_MODEL), lambda g, f, e, m: (e[g], f, 0)),
            pl.BlockSpec((1, 1, _D_MODEL), lambda g, f, e, m: (e[g], 0, 0)),
        ],
        out_specs=pl.BlockSpec((_T, _D_MODEL), lambda g, f, e, m: (g, 0)),
    )
    return pl.pallas_call(
        _moe_body,
        grid_spec=grid_spec,
        out_shape=jax.ShapeDtypeStruct((g_max * _T, _D_MODEL), jnp.float32),
        compiler_params=pltpu.CompilerParams(
            dimension_semantics=("arbitrary", "arbitrary"),
        ),
    )(tile_eid, ntiles, x_pad, W1, b1.reshape(_E, 1, _D_FF), W2,
      b2.reshape(_E, 1, _D_MODEL))


def kernel(inputs, Wg, W1, b1, W2, b2):
    flat = inputs.reshape((-1, inputs.shape[-1]))
    n = flat.shape[0]
    g_max = n // _T + _E - 1

    eid = _gate(flat, Wg)

    counts = jnp.bincount(eid, length=_E)
    start = jnp.concatenate([jnp.zeros((1,), jnp.int32),
                             jnp.cumsum(counts)[:-1].astype(jnp.int32)])
    order = jnp.argsort(eid, stable=True).astype(jnp.int32)
    inv = jnp.zeros((n,), jnp.int32).at[order].set(jnp.arange(n, dtype=jnp.int32))

    tiles_e = (counts + _T - 1) // _T
    cum_tiles = jnp.cumsum(tiles_e).astype(jnp.int32)
    tile_off = cum_tiles - tiles_e.astype(jnp.int32)
    ntiles = cum_tiles[-1]

    gidx = jnp.arange(g_max, dtype=jnp.int32)
    tile_eid = jnp.minimum(
        jnp.sum(gidx[:, None] >= cum_tiles[None, :], axis=1), _E - 1
    ).astype(jnp.int32)

    p = jnp.arange(g_max * _T, dtype=jnp.int32)
    pg = p // _T
    pe = tile_eid[pg]
    rank = (pg - tile_off[pe]) * _T + (p % _T)
    valid = rank < counts[pe]
    src = jnp.where(valid, order[jnp.minimum(start[pe] + rank, n - 1)], 0)

    te = eid
    trank = inv - start[te]
    pos = (tile_off[te] + trank // _T) * _T + trank % _T

    x_pad = jnp.take(flat, src, axis=0)
    y_pad = _grouped_mlp(x_pad, tile_eid, jnp.array([ntiles], jnp.int32),
                         W1, b1, W2, b2, g_max)
    out = jnp.take(y_pad, pos, axis=0)
    return out.reshape(inputs.shape[:-1] + (_D_MODEL,))
